# baseline (speedup 1.0000x reference)
import jax
import jax.numpy as jnp
from jax import lax
from jax.experimental import pallas as pl
from jax.experimental.pallas import tpu as pltpu
from jax.experimental.pallas import tpu_sc as plsc

N = 94
E = 1504
EG = E // 16
NG = 6
F_OUT = 6400
N_TILES = 32
COLS = F_OUT // N_TILES
CG = 13

_i32 = jnp.int32
_f32 = jnp.float32


def _rsqrt(x):
    i = plsc.bitcast(x, _i32)
    i = jnp.full((16,), 0x5F3759DF, _i32) - lax.shift_right_logical(i, 1)
    y = plsc.bitcast(i, _f32)
    for _ in range(3):
        y = y * (1.5 - 0.5 * x * y * y)
    return y


def _tanh(x):
    neg = x < 0.0
    a = jnp.minimum(jnp.abs(x), 10.0)
    z = a * 2.8853900817779268
    k = (z + 0.5).astype(_i32)
    f = z - k.astype(_f32)
    p = 0.0013333558
    p = p * f + 0.009618129
    p = p * f + 0.05550411
    p = p * f + 0.24022652
    p = p * f + 0.6931472
    p = p * f + 1.0
    e2k = plsc.bitcast(lax.shift_left(k + 127, 23), _f32)
    e = p * e2k
    d = e + 1.0
    r = 1.0 / d
    r = r * (2.0 - d * r)
    r = r * (2.0 - d * r)
    t = 1.0 - 2.0 * r
    return jnp.where(neg, -t, t)


def _sc_body(feat_hbm, ei_hbm, ew_hbm, w1_hbm, b1_hbm, w2_hbm, b2_hbm,
             wfc_hbm, bfc_hbm,
             out_hbm,
             wbuf, obuf, bbuf, hloc, hsh,
             featv, w1v, b1v, w2v, b2v, srcv, dstv, ewv, normv,
             degv, dinvv, di2v, xw1, sc1, h1, xw2v, sc2v,
             sem):
    cid = lax.axis_index("c")
    sid = lax.axis_index("s")
    wid = cid * 16 + sid
    base_col = wid * COLS
    iota = lax.iota(_i32, 16)

    def full(v):
        return jnp.full((16,), v, _i32)

    descs = []
    for n in range(N):
        descs.append(pltpu.async_copy(
            wfc_hbm.at[n, pl.ds(base_col, COLS)],
            wbuf.at[pl.ds(n * COLS, COLS)], sem))
    descs.append(pltpu.async_copy(
        bfc_hbm.at[pl.ds(base_col, COLS)], bbuf.at[pl.ds(0, COLS)], sem))

    @pl.when(sid == 0)
    def _gcn():
        pltpu.sync_copy(feat_hbm, featv.at[pl.ds(0, N)])
        pltpu.sync_copy(w1_hbm, w1v)
        pltpu.sync_copy(b1_hbm, b1v)
        pltpu.sync_copy(w2_hbm, w2v)
        pltpu.sync_copy(b2_hbm, b2v)
        pltpu.sync_copy(ei_hbm.at[0], srcv)
        pltpu.sync_copy(ei_hbm.at[1], dstv)
        pltpu.sync_copy(ew_hbm, ewv)

        zvec = lax.shift_right_logical(srcv[pl.ds(0, 16)], 31)

        def w1s(k, j):
            return plsc.load_gather(w1v, [zvec + k, full(j)])

        def b1s(j):
            return plsc.load_gather(b1v, [zvec + j])

        def w2s(j):
            return plsc.load_gather(w2v, [zvec + j, full(0)])

        ones = jnp.ones((16,), _f32)
        zeros = jnp.zeros((16,), _f32)

        for g in range(NG):
            degv[pl.ds(16 * g, 16)] = ones

        def deg_body(i, _):
            off = pl.multiple_of(i * 16, 16)
            d_idx = dstv[pl.ds(off, 16)]
            w = ewv[pl.ds(off, 16)]
            plsc.addupdate_scatter(degv, [d_idx], w)
            return 0

        lax.fori_loop(0, EG, deg_body, 0)

        for g in range(NG):
            d = degv[pl.ds(16 * g, 16)]
            y = _rsqrt(d)
            y = jnp.where(d > 0.0, y, 0.0)
            dinvv[pl.ds(16 * g, 16)] = y
            di2v[pl.ds(16 * g, 16)] = y * y

        def norm_body(i, _):
            off = pl.multiple_of(i * 16, 16)
            s_idx = srcv[pl.ds(off, 16)]
            d_idx = dstv[pl.ds(off, 16)]
            w = ewv[pl.ds(off, 16)]
            a = plsc.load_gather(dinvv, [s_idx])
            b = plsc.load_gather(dinvv, [d_idx])
            normv[pl.ds(off, 16)] = a * w * b
            return 0

        lax.fori_loop(0, EG, norm_body, 0)

        w1b = [[w1s(k, j) for j in range(6)] for k in range(3)]
        for g in range(NG):
            base = iota + 16 * g
            f = [plsc.load_gather(featv, [base, full(k)]) for k in range(3)]
            for j in range(6):
                xw1[j, pl.ds(16 * g, 16)] = (
                    f[0] * w1b[0][j] + f[1] * w1b[1][j] + f[2] * w1b[2][j])

        for j in range(6):
            for g in range(NG):
                sc1[j, pl.ds(16 * g, 16)] = zeros

        def e1_body(i, _):
            off = pl.multiple_of(i * 16, 16)
            s_idx = srcv[pl.ds(off, 16)]
            d_idx = dstv[pl.ds(off, 16)]
            nrm = normv[pl.ds(off, 16)]
            for j in range(6):
                gj = plsc.load_gather(xw1, [full(j), s_idx])
                plsc.addupdate_scatter(sc1, [full(j), d_idx], nrm * gj)
            return 0

        lax.fori_loop(0, EG, e1_body, 0)

        for j in range(6):
            b1j = b1s(j)
            for g in range(NG):
                v = (sc1[j, pl.ds(16 * g, 16)]
                     + di2v[pl.ds(16 * g, 16)] * xw1[j, pl.ds(16 * g, 16)]
                     + b1j)
                h1[j, pl.ds(16 * g, 16)] = _tanh(v)

        w2b = [w2s(j) for j in range(6)]
        for g in range(NG):
            acc = h1[0, pl.ds(16 * g, 16)] * w2b[0]
            for j in range(1, 6):
                acc = acc + h1[j, pl.ds(16 * g, 16)] * w2b[j]
            xw2v[pl.ds(16 * g, 16)] = acc
            sc2v[pl.ds(16 * g, 16)] = zeros

        def e2_body(i, _):
            off = pl.multiple_of(i * 16, 16)
            s_idx = srcv[pl.ds(off, 16)]
            d_idx = dstv[pl.ds(off, 16)]
            nrm = normv[pl.ds(off, 16)]
            gj = plsc.load_gather(xw2v, [s_idx])
            plsc.addupdate_scatter(sc2v, [d_idx], nrm * gj)
            return 0

        lax.fori_loop(0, EG, e2_body, 0)

        b2b = plsc.load_gather(b2v, [zvec])
        for g in range(NG):
            v = (sc2v[pl.ds(16 * g, 16)]
                 + di2v[pl.ds(16 * g, 16)] * xw2v[pl.ds(16 * g, 16)]
                 + b2b)
            hloc[pl.ds(16 * g, 16)] = _tanh(v)
        pltpu.sync_copy(hloc, hsh)

    plsc.subcore_barrier()
    pltpu.sync_copy(hsh, hloc)

    for d in descs:
        d.wait()

    acc0 = tuple(bbuf[pl.ds(16 * g, 16)] for g in range(CG))

    def mv_body(n, accs):
        hb = plsc.load_gather(hloc, [jnp.full((16,), n, _i32)])
        row = pl.multiple_of(n * COLS, 8)
        return tuple(
            accs[g] + hb * wbuf[pl.ds(row + 16 * g, 16)] for g in range(CG))

    accs = lax.fori_loop(0, N, mv_body, acc0)
    for g in range(CG):
        obuf[pl.ds(16 * g, 16)] = accs[g]
    pltpu.sync_copy(obuf.at[pl.ds(0, COLS)], out_hbm.at[pl.ds(base_col, COLS)])


@jax.jit
def _run(feature, ei, ew, w1, b1, w2, b2, wfc, bfc):
    mesh = plsc.VectorSubcoreMesh(core_axis_name="c", subcore_axis_name="s")
    kfn = pl.kernel(
        _sc_body,
        out_type=jax.ShapeDtypeStruct((F_OUT,), _f32),
        mesh=mesh,
        scratch_types=[
            pltpu.VMEM((N * COLS + 16,), _f32),
            pltpu.VMEM((CG * 16,), _f32),
            pltpu.VMEM((CG * 16,), _f32),
            pltpu.VMEM((96,), _f32),
            pltpu.VMEM_SHARED((96,), _f32),
            pltpu.VMEM((96, 3), _f32),
            pltpu.VMEM((3, 6), _f32),
            pltpu.VMEM((6,), _f32),
            pltpu.VMEM((6, 1), _f32),
            pltpu.VMEM((1,), _f32),
            pltpu.VMEM((E,), _i32),
            pltpu.VMEM((E,), _i32),
            pltpu.VMEM((E,), _f32),
            pltpu.VMEM((E,), _f32),
            pltpu.VMEM((96,), _f32),
            pltpu.VMEM((96,), _f32),
            pltpu.VMEM((96,), _f32),
            pltpu.VMEM((6, 96), _f32),
            pltpu.VMEM((6, 96), _f32),
            pltpu.VMEM((6, 96), _f32),
            pltpu.VMEM((96,), _f32),
            pltpu.VMEM((96,), _f32),
            pltpu.SemaphoreType.DMA,
        ],
        compiler_params=pltpu.CompilerParams(
            use_tc_tiling_on_sc=False, needs_layout_passes=False),
        name="gcn_fc_sc",
    )
    return kfn(feature, ei, ew, w1, b1, w2, b2, wfc, bfc)


def kernel(feature, edge_index, edge_weight, W1, b1, W2, b2, Wfc, bfc):
    return _run(feature, edge_index.astype(_i32), edge_weight,
                W1, b1, W2, b2, Wfc, bfc)


# speedup vs baseline: 1.1150x; 1.1150x over previous
import jax
import jax.numpy as jnp
from jax import lax
from jax.experimental import pallas as pl
from jax.experimental.pallas import tpu as pltpu
from jax.experimental.pallas import tpu_sc as plsc

N = 94
E = 1504
EG = E // 16
NG = 6
F_OUT = 6400
N_TILES = 32
COLS = F_OUT // N_TILES
CG = 13

_i32 = jnp.int32
_f32 = jnp.float32


def _rsqrt(x):
    i = plsc.bitcast(x, _i32)
    i = jnp.full((16,), 0x5F3759DF, _i32) - lax.shift_right_logical(i, 1)
    y = plsc.bitcast(i, _f32)
    for _ in range(3):
        y = y * (1.5 - 0.5 * x * y * y)
    return y


def _tanh(x):
    neg = x < 0.0
    a = jnp.minimum(jnp.abs(x), 10.0)
    z = a * 2.8853900817779268
    k = (z + 0.5).astype(_i32)
    f = z - k.astype(_f32)
    p = 0.0013333558
    p = p * f + 0.009618129
    p = p * f + 0.05550411
    p = p * f + 0.24022652
    p = p * f + 0.6931472
    p = p * f + 1.0
    e2k = plsc.bitcast(lax.shift_left(k + 127, 23), _f32)
    e = p * e2k
    d = e + 1.0
    r = 1.0 / d
    r = r * (2.0 - d * r)
    r = r * (2.0 - d * r)
    t = 1.0 - 2.0 * r
    return jnp.where(neg, -t, t)


def _sc_body(feat_hbm, ei_hbm, ew_hbm, w1_hbm, b1_hbm, w2_hbm, b2_hbm,
             wfc_hbm, bfc_hbm,
             out_hbm,
             wbuf, obuf, bbuf, hloc, hsh,
             featv, w1v, b1v, w2v, b2v, srcv, dstv, ewv, normv,
             degv, dinvv, di2v, xw1, sc1, h1, xw2v, sc2v,
             sem, gsem):
    cid = lax.axis_index("c")
    sid = lax.axis_index("s")
    wid = cid * 16 + sid
    base_col = wid * COLS
    iota = lax.iota(_i32, 16)

    def full(v):
        return jnp.full((16,), v, _i32)

    descs = [
        pltpu.async_copy(wfc_hbm.at[:, pl.ds(base_col, COLS)],
                         wbuf.at[:, pl.ds(0, COLS)], sem),
        pltpu.async_copy(bfc_hbm.at[pl.ds(base_col, COLS)],
                         bbuf.at[pl.ds(0, COLS)], sem),
    ]

    @pl.when(sid == 0)
    def _gcn():
        gdescs = [
            pltpu.async_copy(feat_hbm, featv.at[pl.ds(0, N)], gsem),
            pltpu.async_copy(w1_hbm, w1v, gsem),
            pltpu.async_copy(b1_hbm, b1v, gsem),
            pltpu.async_copy(w2_hbm, w2v, gsem),
            pltpu.async_copy(b2_hbm, b2v, gsem),
            pltpu.async_copy(ei_hbm.at[0], srcv, gsem),
            pltpu.async_copy(ei_hbm.at[1], dstv, gsem),
            pltpu.async_copy(ew_hbm, ewv, gsem),
        ]
        for d in gdescs:
            d.wait()

        zvec = lax.shift_right_logical(srcv[pl.ds(0, 16)], 31)

        def w1s(k, j):
            return plsc.load_gather(w1v, [zvec + k, full(j)])

        def b1s(j):
            return plsc.load_gather(b1v, [zvec + j])

        def w2s(j):
            return plsc.load_gather(w2v, [zvec + j, full(0)])

        ones = jnp.ones((16,), _f32)
        zeros = jnp.zeros((16,), _f32)

        for g in range(NG):
            degv[pl.ds(16 * g, 16)] = ones

        def deg_body(i, _):
            off = pl.multiple_of(i * 16, 16)
            d_idx = dstv[pl.ds(off, 16)]
            w = ewv[pl.ds(off, 16)]
            plsc.addupdate_scatter(degv, [d_idx], w)
            return 0

        lax.fori_loop(0, EG, deg_body, 0)

        for g in range(NG):
            d = degv[pl.ds(16 * g, 16)]
            y = _rsqrt(d)
            y = jnp.where(d > 0.0, y, 0.0)
            dinvv[pl.ds(16 * g, 16)] = y
            di2v[pl.ds(16 * g, 16)] = y * y

        def norm_body(i, _):
            off = pl.multiple_of(i * 16, 16)
            s_idx = srcv[pl.ds(off, 16)]
            d_idx = dstv[pl.ds(off, 16)]
            w = ewv[pl.ds(off, 16)]
            a = plsc.load_gather(dinvv, [s_idx])
            b = plsc.load_gather(dinvv, [d_idx])
            normv[pl.ds(off, 16)] = a * w * b
            return 0

        lax.fori_loop(0, EG, norm_body, 0)

        w1b = [[w1s(k, j) for j in range(6)] for k in range(3)]
        for g in range(NG):
            base = iota + 16 * g
            f = [plsc.load_gather(featv, [base, full(k)]) for k in range(3)]
            for j in range(6):
                xw1[j, pl.ds(16 * g, 16)] = (
                    f[0] * w1b[0][j] + f[1] * w1b[1][j] + f[2] * w1b[2][j])

        for j in range(6):
            for g in range(NG):
                sc1[j, pl.ds(16 * g, 16)] = zeros

        def e1_body(i, _):
            off = pl.multiple_of(i * 16, 16)
            s_idx = srcv[pl.ds(off, 16)]
            d_idx = dstv[pl.ds(off, 16)]
            nrm = normv[pl.ds(off, 16)]
            for j in range(6):
                gj = plsc.load_gather(xw1, [full(j), s_idx])
                plsc.addupdate_scatter(sc1, [full(j), d_idx], nrm * gj)
            return 0

        lax.fori_loop(0, EG, e1_body, 0)

        for j in range(6):
            b1j = b1s(j)
            for g in range(NG):
                v = (sc1[j, pl.ds(16 * g, 16)]
                     + di2v[pl.ds(16 * g, 16)] * xw1[j, pl.ds(16 * g, 16)]
                     + b1j)
                h1[j, pl.ds(16 * g, 16)] = _tanh(v)

        w2b = [w2s(j) for j in range(6)]
        for g in range(NG):
            acc = h1[0, pl.ds(16 * g, 16)] * w2b[0]
            for j in range(1, 6):
                acc = acc + h1[j, pl.ds(16 * g, 16)] * w2b[j]
            xw2v[pl.ds(16 * g, 16)] = acc
            sc2v[pl.ds(16 * g, 16)] = zeros

        def e2_body(i, _):
            off = pl.multiple_of(i * 16, 16)
            s_idx = srcv[pl.ds(off, 16)]
            d_idx = dstv[pl.ds(off, 16)]
            nrm = normv[pl.ds(off, 16)]
            gj = plsc.load_gather(xw2v, [s_idx])
            plsc.addupdate_scatter(sc2v, [d_idx], nrm * gj)
            return 0

        lax.fori_loop(0, EG, e2_body, 0)

        b2b = plsc.load_gather(b2v, [zvec])
        for g in range(NG):
            v = (sc2v[pl.ds(16 * g, 16)]
                 + di2v[pl.ds(16 * g, 16)] * xw2v[pl.ds(16 * g, 16)]
                 + b2b)
            hloc[pl.ds(16 * g, 16)] = _tanh(v)
        pltpu.sync_copy(hloc, hsh)

    plsc.subcore_barrier()
    pltpu.sync_copy(hsh, hloc)

    for d in descs:
        d.wait()

    acc0 = tuple(bbuf[pl.ds(16 * g, 16)] for g in range(CG))

    def mv_body(n, accs):
        hb = plsc.load_gather(hloc, [jnp.full((16,), n, _i32)])
        return tuple(
            accs[g] + hb * wbuf[n, pl.ds(16 * g, 16)] for g in range(CG))

    accs = lax.fori_loop(0, N, mv_body, acc0)
    for g in range(CG):
        obuf[pl.ds(16 * g, 16)] = accs[g]
    pltpu.sync_copy(obuf.at[pl.ds(0, COLS)], out_hbm.at[pl.ds(base_col, COLS)])


@jax.jit
def _run(feature, ei, ew, w1, b1, w2, b2, wfc, bfc):
    mesh = plsc.VectorSubcoreMesh(core_axis_name="c", subcore_axis_name="s")
    kfn = pl.kernel(
        _sc_body,
        out_type=jax.ShapeDtypeStruct((F_OUT,), _f32),
        mesh=mesh,
        scratch_types=[
            pltpu.VMEM((N, CG * 16), _f32),
            pltpu.VMEM((CG * 16,), _f32),
            pltpu.VMEM((CG * 16,), _f32),
            pltpu.VMEM((96,), _f32),
            pltpu.VMEM_SHARED((96,), _f32),
            pltpu.VMEM((96, 3), _f32),
            pltpu.VMEM((3, 6), _f32),
            pltpu.VMEM((6,), _f32),
            pltpu.VMEM((6, 1), _f32),
            pltpu.VMEM((1,), _f32),
            pltpu.VMEM((E,), _i32),
            pltpu.VMEM((E,), _i32),
            pltpu.VMEM((E,), _f32),
            pltpu.VMEM((E,), _f32),
            pltpu.VMEM((96,), _f32),
            pltpu.VMEM((96,), _f32),
            pltpu.VMEM((96,), _f32),
            pltpu.VMEM((6, 96), _f32),
            pltpu.VMEM((6, 96), _f32),
            pltpu.VMEM((6, 96), _f32),
            pltpu.VMEM((96,), _f32),
            pltpu.VMEM((96,), _f32),
            pltpu.SemaphoreType.DMA,
            pltpu.SemaphoreType.DMA,
        ],
        compiler_params=pltpu.CompilerParams(
            use_tc_tiling_on_sc=False, needs_layout_passes=False),
        name="gcn_fc_sc",
    )
    return kfn(feature, ei, ew, w1, b1, w2, b2, wfc, bfc)


def kernel(feature, edge_index, edge_weight, W1, b1, W2, b2, Wfc, bfc):
    return _run(feature, edge_index.astype(_i32), edge_weight,
                W1, b1, W2, b2, Wfc, bfc)


# speedup vs baseline: 1.2412x; 1.1132x over previous
import jax
import jax.numpy as jnp
from jax import lax
from jax.experimental import pallas as pl
from jax.experimental.pallas import tpu as pltpu
from jax.experimental.pallas import tpu_sc as plsc

N = 94
E = 1504
EG = E // 16
NG = 6
F_OUT = 6400
N_TILES = 25
COLS = F_OUT // N_TILES
CG = COLS // 16

_i32 = jnp.int32
_f32 = jnp.float32


def _rsqrt(x):
    i = plsc.bitcast(x, _i32)
    i = jnp.full((16,), 0x5F3759DF, _i32) - lax.shift_right_logical(i, 1)
    y = plsc.bitcast(i, _f32)
    for _ in range(3):
        y = y * (1.5 - 0.5 * x * y * y)
    return y


def _tanh(x):
    neg = x < 0.0
    a = jnp.minimum(jnp.abs(x), 10.0)
    z = a * 2.8853900817779268
    k = (z + 0.5).astype(_i32)
    f = z - k.astype(_f32)
    p = 0.0013333558
    p = p * f + 0.009618129
    p = p * f + 0.05550411
    p = p * f + 0.24022652
    p = p * f + 0.6931472
    p = p * f + 1.0
    e2k = plsc.bitcast(lax.shift_left(k + 127, 23), _f32)
    e = p * e2k
    d = e + 1.0
    r = 1.0 / d
    r = r * (2.0 - d * r)
    r = r * (2.0 - d * r)
    t = 1.0 - 2.0 * r
    return jnp.where(neg, -t, t)


def _sc_body(feat_hbm, ei_hbm, ew_hbm, w1_hbm, b1_hbm, w2_hbm, b2_hbm,
             wfc_hbm, bfc_hbm,
             out_hbm,
             wbuf, obuf, bbuf, hloc, hsh,
             featv, w1v, b1v, w2v, b2v, srcv, dstv, ewv, normv,
             degv, dinvv, di2v, xw1, sc1, h1, xw2v, sc2v,
             sem, gsem):
    cid = lax.axis_index("c")
    sid = lax.axis_index("s")
    wid = cid * 16 + sid
    base_col = wid * COLS
    iota = lax.iota(_i32, 16)

    def full(v):
        return jnp.full((16,), v, _i32)

    @pl.when(wid < N_TILES)
    def _fire():
        pltpu.async_copy(wfc_hbm.at[:, pl.ds(base_col, COLS)], wbuf, sem)
        pltpu.async_copy(bfc_hbm.at[pl.ds(base_col, COLS)], bbuf, sem)

    @pl.when(sid == 0)
    def _gcn():
        gdescs = [
            pltpu.async_copy(feat_hbm, featv.at[pl.ds(0, N)], gsem),
            pltpu.async_copy(w1_hbm, w1v, gsem),
            pltpu.async_copy(b1_hbm, b1v, gsem),
            pltpu.async_copy(w2_hbm, w2v, gsem),
            pltpu.async_copy(b2_hbm, b2v, gsem),
            pltpu.async_copy(ei_hbm.at[0], srcv, gsem),
            pltpu.async_copy(ei_hbm.at[1], dstv, gsem),
            pltpu.async_copy(ew_hbm, ewv, gsem),
        ]
        for d in gdescs:
            d.wait()

        zvec = lax.shift_right_logical(srcv[pl.ds(0, 16)], 31)

        def w1s(k, j):
            return plsc.load_gather(w1v, [zvec + k, full(j)])

        def b1s(j):
            return plsc.load_gather(b1v, [zvec + j])

        def w2s(j):
            return plsc.load_gather(w2v, [zvec + j, full(0)])

        ones = jnp.ones((16,), _f32)
        zeros = jnp.zeros((16,), _f32)

        for g in range(NG):
            degv[pl.ds(16 * g, 16)] = ones

        def deg_body(i, _):
            off = pl.multiple_of(i * 16, 16)
            d_idx = dstv[pl.ds(off, 16)]
            w = ewv[pl.ds(off, 16)]
            plsc.addupdate_scatter(degv, [d_idx], w)
            return 0

        lax.fori_loop(0, EG, deg_body, 0)

        for g in range(NG):
            d = degv[pl.ds(16 * g, 16)]
            y = _rsqrt(d)
            y = jnp.where(d > 0.0, y, 0.0)
            dinvv[pl.ds(16 * g, 16)] = y
            di2v[pl.ds(16 * g, 16)] = y * y

        def norm_body(i, _):
            off = pl.multiple_of(i * 16, 16)
            s_idx = srcv[pl.ds(off, 16)]
            d_idx = dstv[pl.ds(off, 16)]
            w = ewv[pl.ds(off, 16)]
            a = plsc.load_gather(dinvv, [s_idx])
            b = plsc.load_gather(dinvv, [d_idx])
            normv[pl.ds(off, 16)] = a * w * b
            return 0

        lax.fori_loop(0, EG, norm_body, 0)

        w1b = [[w1s(k, j) for j in range(6)] for k in range(3)]
        for g in range(NG):
            base = iota + 16 * g
            f = [plsc.load_gather(featv, [base, full(k)]) for k in range(3)]
            for j in range(6):
                xw1[j, pl.ds(16 * g, 16)] = (
                    f[0] * w1b[0][j] + f[1] * w1b[1][j] + f[2] * w1b[2][j])

        for j in range(6):
            for g in range(NG):
                sc1[j, pl.ds(16 * g, 16)] = zeros

        def e1_body(i, _):
            off = pl.multiple_of(i * 16, 16)
            s_idx = srcv[pl.ds(off, 16)]
            d_idx = dstv[pl.ds(off, 16)]
            nrm = normv[pl.ds(off, 16)]
            for j in range(6):
                gj = plsc.load_gather(xw1, [full(j), s_idx])
                plsc.addupdate_scatter(sc1, [full(j), d_idx], nrm * gj)
            return 0

        lax.fori_loop(0, EG, e1_body, 0)

        for j in range(6):
            b1j = b1s(j)
            for g in range(NG):
                v = (sc1[j, pl.ds(16 * g, 16)]
                     + di2v[pl.ds(16 * g, 16)] * xw1[j, pl.ds(16 * g, 16)]
                     + b1j)
                h1[j, pl.ds(16 * g, 16)] = _tanh(v)

        w2b = [w2s(j) for j in range(6)]
        for g in range(NG):
            acc = h1[0, pl.ds(16 * g, 16)] * w2b[0]
            for j in range(1, 6):
                acc = acc + h1[j, pl.ds(16 * g, 16)] * w2b[j]
            xw2v[pl.ds(16 * g, 16)] = acc
            sc2v[pl.ds(16 * g, 16)] = zeros

        def e2_body(i, _):
            off = pl.multiple_of(i * 16, 16)
            s_idx = srcv[pl.ds(off, 16)]
            d_idx = dstv[pl.ds(off, 16)]
            nrm = normv[pl.ds(off, 16)]
            gj = plsc.load_gather(xw2v, [s_idx])
            plsc.addupdate_scatter(sc2v, [d_idx], nrm * gj)
            return 0

        lax.fori_loop(0, EG, e2_body, 0)

        b2b = plsc.load_gather(b2v, [zvec])
        for g in range(NG):
            v = (sc2v[pl.ds(16 * g, 16)]
                 + di2v[pl.ds(16 * g, 16)] * xw2v[pl.ds(16 * g, 16)]
                 + b2b)
            hloc[pl.ds(16 * g, 16)] = _tanh(v)
        pltpu.sync_copy(hloc, hsh)

    plsc.subcore_barrier()
    pltpu.sync_copy(hsh, hloc)

    @pl.when(wid < N_TILES)
    def _matvec():
        pltpu.make_async_copy(wfc_hbm.at[:, pl.ds(base_col, COLS)], wbuf,
                              sem).wait()
        pltpu.make_async_copy(bfc_hbm.at[pl.ds(base_col, COLS)], bbuf,
                              sem).wait()
        acc0 = tuple(bbuf[pl.ds(16 * g, 16)] for g in range(CG))

        def mv_body(n, accs):
            hb = plsc.load_gather(hloc, [jnp.full((16,), n, _i32)])
            return tuple(
                accs[g] + hb * wbuf[n, pl.ds(16 * g, 16)] for g in range(CG))

        accs = lax.fori_loop(0, N, mv_body, acc0)
        for g in range(CG):
            obuf[pl.ds(16 * g, 16)] = accs[g]
        pltpu.sync_copy(obuf, out_hbm.at[pl.ds(base_col, COLS)])


@jax.jit
def _run(feature, ei, ew, w1, b1, w2, b2, wfc, bfc):
    mesh = plsc.VectorSubcoreMesh(core_axis_name="c", subcore_axis_name="s")
    kfn = pl.kernel(
        _sc_body,
        out_type=jax.ShapeDtypeStruct((F_OUT,), _f32),
        mesh=mesh,
        scratch_types=[
            pltpu.VMEM((N, COLS), _f32),
            pltpu.VMEM((COLS,), _f32),
            pltpu.VMEM((COLS,), _f32),
            pltpu.VMEM((96,), _f32),
            pltpu.VMEM_SHARED((96,), _f32),
            pltpu.VMEM((96, 3), _f32),
            pltpu.VMEM((3, 6), _f32),
            pltpu.VMEM((6,), _f32),
            pltpu.VMEM((6, 1), _f32),
            pltpu.VMEM((1,), _f32),
            pltpu.VMEM((E,), _i32),
            pltpu.VMEM((E,), _i32),
            pltpu.VMEM((E,), _f32),
            pltpu.VMEM((E,), _f32),
            pltpu.VMEM((96,), _f32),
            pltpu.VMEM((96,), _f32),
            pltpu.VMEM((96,), _f32),
            pltpu.VMEM((6, 96), _f32),
            pltpu.VMEM((6, 96), _f32),
            pltpu.VMEM((6, 96), _f32),
            pltpu.VMEM((96,), _f32),
            pltpu.VMEM((96,), _f32),
            pltpu.SemaphoreType.DMA,
            pltpu.SemaphoreType.DMA,
        ],
        compiler_params=pltpu.CompilerParams(
            use_tc_tiling_on_sc=True, needs_layout_passes=False),
        name="gcn_fc_sc",
    )
    return kfn(feature, ei, ew, w1, b1, w2, b2, wfc, bfc)


def kernel(feature, edge_index, edge_weight, W1, b1, W2, b2, Wfc, bfc):
    return _run(feature, edge_index.astype(_i32), edge_weight,
                W1, b1, W2, b2, Wfc, bfc)
